# baseline (speedup 1.0000x reference)
import functools

import jax
import jax.numpy as jnp
from jax import lax
from jax.experimental import pallas as pl
from jax.experimental.pallas import tpu as pltpu
from jax.experimental.pallas import tpu_sc as plsc

_T = 500000
_H = 128
_NCLS = 10
_THR = 4.0
_CLIP = 100.0

_NC = 2
_NS = 16
_NW = _NC * _NS

_RPW = 4375
_RSC = _NW * _RPW
_TTC = _T - _RSC
_CSC = 125
_NCH = _RPW // _CSC

_BR = 24000
_NBTC = _TTC // _BR
_BRO = 25000
_NBO = _T // _BRO

_S8 = jax.ShapeDtypeStruct((8, _H), jnp.float32)
_SW = jax.ShapeDtypeStruct((_NW, _H), jnp.float32)
_SWF = jax.ShapeDtypeStruct((_NW * _H,), jnp.float32)
_stat_spec = pl.BlockSpec((8, _H), lambda i: (0, 0))
_statw_spec = pl.BlockSpec((_NW, _H), lambda i: (0, 0))
_params = pltpu.CompilerParams(dimension_semantics=("arbitrary",))


def _colsum(a):
    return jnp.sum(a, axis=0, keepdims=True)


def _mean_invstd(s, q, n):
    m = s / n
    v = jnp.maximum((q - n * m * m) / (n - 1.0), 0.0)
    sd = jnp.maximum(jnp.sqrt(v), 1e-6)
    return m, sd


def _p1_body(x_ref, s_ref, q_ref):
    @pl.when(pl.program_id(0) == 0)
    def _():
        s_ref[...] = jnp.zeros_like(s_ref)
        q_ref[...] = jnp.zeros_like(q_ref)

    x3 = x_ref[...].reshape(_BR // 8, 8, _H)
    s_ref[...] += jnp.sum(x3, axis=0)
    q_ref[...] += jnp.sum(x3 * x3, axis=0)


def _p2_body(x_ref, lo_ref, hi_ref, ms_ref, mq_ref, mc_ref):
    @pl.when(pl.program_id(0) == 0)
    def _():
        ms_ref[...] = jnp.zeros_like(ms_ref)
        mq_ref[...] = jnp.zeros_like(mq_ref)
        mc_ref[...] = jnp.zeros_like(mc_ref)

    x3 = x_ref[...].reshape(_BR // 8, 8, _H)
    lo, hi = lo_ref[...], hi_ref[...]
    msk = (x3 >= lo) & (x3 <= hi)
    xm = jnp.where(msk, x3, 0.0)
    ms_ref[...] += jnp.sum(xm, axis=0)
    mq_ref[...] += jnp.sum(xm * xm, axis=0)
    mc_ref[...] += jnp.sum(msk.astype(jnp.float32), axis=0)


def _p3_body(x_ref, lo_ref, hi_ref, cs_ref, cq_ref):
    @pl.when(pl.program_id(0) == 0)
    def _():
        cs_ref[...] = jnp.zeros_like(cs_ref)
        cq_ref[...] = jnp.zeros_like(cq_ref)

    x3 = x_ref[...].reshape(_BR // 8, 8, _H)
    xc = jnp.clip(x3, lo_ref[...], hi_ref[...])
    cs_ref[...] += jnp.sum(xc, axis=0)
    cq_ref[...] += jnp.sum(xc * xc, axis=0)


def _p4_body(x_ref, lo_ref, hi_ref, m_ref, r_ref, o_ref):
    x3 = x_ref[...].reshape(_BRO // 8, 8, _H)
    xc = jnp.clip(x3, lo_ref[...], hi_ref[...])
    o3 = jnp.clip((xc - m_ref[...]) * r_ref[...], -_CLIP, _CLIP)
    o_ref[...] = o3.reshape(_BRO, _H)


def _k1_body(s_tc, q_tc, s_sc, q_sc, lo_ref, hi_ref):
    s = _colsum(s_tc[...]) + _colsum(s_sc[...])
    q = _colsum(q_tc[...]) + _colsum(q_sc[...])
    m, sd = _mean_invstd(s, q, float(_T))
    lo_ref[...] = jnp.broadcast_to(m - _THR * sd, (8, _H))
    hi_ref[...] = jnp.broadcast_to(m + _THR * sd, (8, _H))


def _k2_body(ms_tc, mq_tc, mc_tc, ms_sc, mq_sc, mc_sc, lo_ref, hi_ref):
    s = _colsum(ms_tc[...]) + _colsum(ms_sc[...])
    q = _colsum(mq_tc[...]) + _colsum(mq_sc[...])
    c = _colsum(mc_tc[...]) + _colsum(mc_sc[...])
    m, sd = _mean_invstd(s, q, c)
    lo_ref[...] = jnp.broadcast_to(m - _THR * sd, (8, _H))
    hi_ref[...] = jnp.broadcast_to(m + _THR * sd, (8, _H))


def _k3_body(cs_tc, cq_tc, cs_sc, cq_sc, m_ref, r_ref):
    s = _colsum(cs_tc[...]) + _colsum(cs_sc[...])
    q = _colsum(cq_tc[...]) + _colsum(cq_sc[...])
    m, sd = _mean_invstd(s, q, float(_T))
    m_ref[...] = jnp.broadcast_to(m, (8, _H))
    r_ref[...] = jnp.broadcast_to(1.0 / sd, (8, _H))


_x_spec = pl.BlockSpec((_BR, _H), lambda i: (i, 0))
_xo_spec = pl.BlockSpec((_BRO, _H), lambda i: (i, 0))


def _run_p1(x_tc):
    return pl.pallas_call(
        _p1_body, grid=(_NBTC,),
        in_specs=[_x_spec],
        out_specs=(_stat_spec, _stat_spec),
        out_shape=(_S8, _S8),
        compiler_params=_params,
    )(x_tc)


def _run_p2(x_tc, lo, hi):
    return pl.pallas_call(
        _p2_body, grid=(_NBTC,),
        in_specs=[_x_spec, _stat_spec, _stat_spec],
        out_specs=(_stat_spec, _stat_spec, _stat_spec),
        out_shape=(_S8, _S8, _S8),
        compiler_params=_params,
    )(x_tc, lo, hi)


def _run_p3(x_tc, lo, hi):
    return pl.pallas_call(
        _p3_body, grid=(_NBTC,),
        in_specs=[_x_spec, _stat_spec, _stat_spec],
        out_specs=(_stat_spec, _stat_spec),
        out_shape=(_S8, _S8),
        compiler_params=_params,
    )(x_tc, lo, hi)


def _run_p4(x, lo, hi, m, r):
    return pl.pallas_call(
        _p4_body, grid=(_NBO,),
        in_specs=[_xo_spec] + [_stat_spec] * 4,
        out_specs=_xo_spec,
        out_shape=jax.ShapeDtypeStruct((_T, _H), jnp.float32),
        compiler_params=_params,
    )(x, lo, hi, m, r)


def _run_k(body, n_in, n_out, *args):
    return pl.pallas_call(
        body, grid=(1,),
        in_specs=[_stat_spec if a.shape == (8, _H) else _statw_spec
                  for a in args],
        out_specs=tuple([_stat_spec] * n_out),
        out_shape=tuple([_S8] * n_out),
        compiler_params=_params,
    )(*args)


def _sc_mesh():
    return plsc.VectorSubcoreMesh(core_axis_name="c", subcore_axis_name="s")


def _worker_id():
    return lax.axis_index("s") * _NC + lax.axis_index("c")


def _row_loop(xbuf, accs, step):
    return lax.fori_loop(0, _CSC, step, accs)


def _sc_p1_body(x_hbm, s_out, q_out, xbuf, obuf):
    w = _worker_id()
    base = (_TTC + w * _RPW) * _H

    def chunk(c, accs):
        pltpu.sync_copy(x_hbm.at[pl.ds(base + c * (_CSC * _H), _CSC * _H)],
                        xbuf)

        def row(i, a):
            new = list(a)
            for k in range(8):
                v = xbuf[pl.ds(i * _H + k * 16, 16)]
                new[k] = a[k] + v
                new[8 + k] = a[8 + k] + v * v
            return tuple(new)

        return lax.fori_loop(0, _CSC, row, accs)

    zero = jnp.zeros((16,), jnp.float32)
    accs = lax.fori_loop(0, _NCH, chunk, (zero,) * 16)
    for k in range(8):
        obuf[pl.ds(16 * k, 16)] = accs[k]
    pltpu.sync_copy(obuf, s_out.at[pl.ds(w * _H, _H)])
    for k in range(8):
        obuf[pl.ds(16 * k, 16)] = accs[8 + k]
    pltpu.sync_copy(obuf, q_out.at[pl.ds(w * _H, _H)])


def _sc_p2_body(x_hbm, lo_hbm, hi_hbm, ms_out, mq_out, mc_out, xbuf, pbuf,
                obuf):
    w = _worker_id()
    base = (_TTC + w * _RPW) * _H
    pltpu.sync_copy(lo_hbm.at[pl.ds(0, _H)], pbuf)
    los = [pbuf[pl.ds(16 * k, 16)] for k in range(8)]
    pltpu.sync_copy(hi_hbm.at[pl.ds(0, _H)], obuf)
    his = [obuf[pl.ds(16 * k, 16)] for k in range(8)]

    def chunk(c, accs):
        pltpu.sync_copy(x_hbm.at[pl.ds(base + c * (_CSC * _H), _CSC * _H)],
                        xbuf)

        def row(i, a):
            new = list(a)
            for k in range(8):
                v = xbuf[pl.ds(i * _H + k * 16, 16)]
                m = (v >= los[k]) & (v <= his[k])
                xm = jnp.where(m, v, 0.0)
                new[k] = a[k] + xm
                new[8 + k] = a[8 + k] + xm * xm
                new[16 + k] = a[16 + k] + jnp.where(m, 1.0, 0.0)
            return tuple(new)

        return lax.fori_loop(0, _CSC, row, accs)

    zero = jnp.zeros((16,), jnp.float32)
    accs = lax.fori_loop(0, _NCH, chunk, (zero,) * 24)
    for g, out in ((0, ms_out), (8, mq_out), (16, mc_out)):
        for k in range(8):
            obuf[pl.ds(16 * k, 16)] = accs[g + k]
        pltpu.sync_copy(obuf, out.at[pl.ds(w * _H, _H)])


def _sc_p3_body(x_hbm, lo_hbm, hi_hbm, cs_out, cq_out, xbuf, pbuf, obuf):
    w = _worker_id()
    base = (_TTC + w * _RPW) * _H
    pltpu.sync_copy(lo_hbm.at[pl.ds(0, _H)], pbuf)
    los = [pbuf[pl.ds(16 * k, 16)] for k in range(8)]
    pltpu.sync_copy(hi_hbm.at[pl.ds(0, _H)], obuf)
    his = [obuf[pl.ds(16 * k, 16)] for k in range(8)]

    def chunk(c, accs):
        pltpu.sync_copy(x_hbm.at[pl.ds(base + c * (_CSC * _H), _CSC * _H)],
                        xbuf)

        def row(i, a):
            new = list(a)
            for k in range(8):
                v = xbuf[pl.ds(i * _H + k * 16, 16)]
                xc = jnp.minimum(jnp.maximum(v, los[k]), his[k])
                new[k] = a[k] + xc
                new[8 + k] = a[8 + k] + xc * xc
            return tuple(new)

        return lax.fori_loop(0, _CSC, row, accs)

    zero = jnp.zeros((16,), jnp.float32)
    accs = lax.fori_loop(0, _NCH, chunk, (zero,) * 16)
    for g, out in ((0, cs_out), (8, cq_out)):
        for k in range(8):
            obuf[pl.ds(16 * k, 16)] = accs[g + k]
        pltpu.sync_copy(obuf, out.at[pl.ds(w * _H, _H)])


def _build_sc_p1():
    return functools.partial(
        pl.kernel, mesh=_sc_mesh(),
        out_type=(_SWF, _SWF),
        scratch_types=[
            pltpu.VMEM((_CSC * _H,), jnp.float32),
            pltpu.VMEM((_H,), jnp.float32),
        ],
    )(_sc_p1_body)


def _build_sc_p2():
    return functools.partial(
        pl.kernel, mesh=_sc_mesh(),
        out_type=(_SWF, _SWF, _SWF),
        scratch_types=[
            pltpu.VMEM((_CSC * _H,), jnp.float32),
            pltpu.VMEM((_H,), jnp.float32),
            pltpu.VMEM((_H,), jnp.float32),
        ],
    )(_sc_p2_body)


def _build_sc_p3():
    return functools.partial(
        pl.kernel, mesh=_sc_mesh(),
        out_type=(_SWF, _SWF),
        scratch_types=[
            pltpu.VMEM((_CSC * _H,), jnp.float32),
            pltpu.VMEM((_H,), jnp.float32),
            pltpu.VMEM((_H,), jnp.float32),
        ],
    )(_sc_p3_body)


_YB = 2000
_NYB = _T // _YB
_BPW = -(-_NYB // _NW)


def _build_labels_sc():
    return functools.partial(
        pl.kernel, mesh=_sc_mesh(),
        out_type=jax.ShapeDtypeStruct((_T,), jnp.int32),
        scratch_types=[
            pltpu.VMEM((16 * (_NCLS - 1),), jnp.int32),
            pltpu.VMEM((16 * (_NCLS - 1),), jnp.float32),
            pltpu.VMEM((_YB,), jnp.float32),
            pltpu.VMEM((_YB,), jnp.int32),
            pltpu.SemaphoreType.DMA,
        ],
    )(_labels_sc_body)


def _labels_sc_body(y_hbm, idx_hbm, out_hbm, idx_v, b_v, y_v, o_v, sem):
    wid = _worker_id()
    pltpu.sync_copy(idx_hbm, idx_v)
    pltpu.async_copy(y_hbm.at[idx_v], b_v, sem).wait()
    bvecs = [b_v[pl.ds(16 * j, 16)] for j in range(_NCLS - 1)]

    for t in range(_BPW):
        blk = wid + t * _NW

        @pl.when(blk < _NYB)
        def _():
            base = blk * _YB
            pltpu.sync_copy(y_hbm.at[pl.ds(base, _YB)], y_v)

            def body(i, carry):
                v = y_v[pl.ds(i * 16, 16)]
                acc = jnp.zeros((16,), jnp.int32)
                for bj in bvecs:
                    acc = acc + jnp.where(v > bj, 1, 0)
                o_v[pl.ds(i * 16, 16)] = acc
                return carry

            lax.fori_loop(0, _YB // 16, body, 0)
            pltpu.sync_copy(o_v, out_hbm.at[pl.ds(base, _YB)])


def kernel(x, y):
    x_tc = x
    x_flat = x.reshape(_T * _H)

    s_tc, q_tc = _run_p1(x_tc)
    s_sc, q_sc = _build_sc_p1()(x_flat)
    s_sc = s_sc.reshape(_NW, _H)
    q_sc = q_sc.reshape(_NW, _H)
    lo1, hi1 = _run_k(_k1_body, 4, 2, s_tc, q_tc, s_sc, q_sc)

    ms_tc, mq_tc, mc_tc = _run_p2(x_tc, lo1, hi1)
    ms_sc, mq_sc, mc_sc = _build_sc_p2()(x_flat, lo1.reshape(8 * _H),
                                         hi1.reshape(8 * _H))
    ms_sc = ms_sc.reshape(_NW, _H)
    mq_sc = mq_sc.reshape(_NW, _H)
    mc_sc = mc_sc.reshape(_NW, _H)
    lo2, hi2 = _run_k(_k2_body, 6, 2, ms_tc, mq_tc, mc_tc, ms_sc, mq_sc,
                      mc_sc)

    cs_tc, cq_tc = _run_p3(x_tc, lo2, hi2)
    cs_sc, cq_sc = _build_sc_p3()(x_flat, lo2.reshape(8 * _H),
                                  hi2.reshape(8 * _H))
    cs_sc = cs_sc.reshape(_NW, _H)
    cq_sc = cq_sc.reshape(_NW, _H)
    m2, r2 = _run_k(_k3_body, 4, 2, cs_tc, cq_tc, cs_sc, cq_sc)

    x_proc = _run_p4(x, lo2, hi2, m2, r2)

    bidx = jax.random.randint(jax.random.key(42), (_NCLS - 1,), 0, _T)
    idx_rep = jnp.repeat(bidx.astype(jnp.int32), 16)
    labels = _build_labels_sc()(y, idx_rep)
    return x_proc, labels
